# baseline (device time: 110028 ns/iter reference)
import jax
import jax.numpy as jnp
from jax import lax
from jax.experimental import pallas as pl
from jax.experimental.pallas import tpu as pltpu

N_DEV = 4


def kernel(x, W1, W2):
    W1 = W1.astype(jnp.bfloat16)
    W2 = W2.astype(jnp.bfloat16)
    m, _ = x.shape
    out_n = W2.shape[1]
    ch = m // N_DEV
    q = ch // 4

    OFF_RA, OFF_RB, OFF_LA, OFF_LB = 0, q, 2 * q, 3 * q

    def body(x_ref, w1_ref, w2_ref, out_ref,
             comm_ar, comm_al, comm_br, comm_bl,
             send_ar, recv_ar, send_al, recv_al,
             send_br, recv_br, send_bl, recv_bl):
        my_pos = lax.axis_index("i")
        left = (my_pos + N_DEV - 1) % N_DEV
        right = (my_pos + 1) % N_DEV

        def compute_q(c, off):
            xs = x_ref[pl.ds(c * ch + off, q), :].astype(jnp.bfloat16)
            h = jnp.dot(xs, w1_ref[...], preferred_element_type=jnp.float32)
            h = jnp.maximum(h, 0.0).astype(jnp.bfloat16)
            return jnp.dot(h, w2_ref[...], preferred_element_type=jnp.float32)

        def rs_hop(comm, ssem, rsem, k, nbr):
            ss, rs = k % 2, (k + 1) % 2
            return pltpu.make_async_remote_copy(
                src_ref=comm.at[ss], dst_ref=comm.at[rs],
                send_sem=ssem.at[ss], recv_sem=rsem.at[rs],
                device_id=(nbr,), device_id_type=pl.DeviceIdType.MESH,
            )

        def ag_hop(c, off, ssem, rsem, k, nbr):
            ss, rs = k % 2, (k + 1) % 2
            region = out_ref.at[pl.ds(c * ch + off, q)]
            return pltpu.make_async_remote_copy(
                src_ref=region, dst_ref=region,
                send_sem=ssem.at[ss], recv_sem=rsem.at[rs],
                device_id=(nbr,), device_id_type=pl.DeviceIdType.MESH,
            )

        comm_ar[0, :, :] = compute_q(my_pos, OFF_RA).astype(jnp.bfloat16)
        comm_al[0, :, :] = compute_q(my_pos, OFF_LA).astype(jnp.bfloat16)

        barrier_sem = pltpu.get_barrier_semaphore()
        for nbr in [left, right]:
            pl.semaphore_signal(
                barrier_sem, inc=1,
                device_id=(nbr,), device_id_type=pl.DeviceIdType.MESH,
            )
        pl.semaphore_wait(barrier_sem, 2)

        for k in range(N_DEV - 1):
            rdma_r = rs_hop(comm_ar, send_ar, recv_ar, k, right)
            rdma_l = rs_hop(comm_al, send_al, recv_al, k, left)
            rdma_r.start()
            rdma_l.start()
            c_r = (my_pos + (N_DEV - 1 - k)) % N_DEV
            c_l = (my_pos + k + 1) % N_DEV
            p_r = compute_q(c_r, OFF_RA)
            p_l = compute_q(c_l, OFF_LA)
            if k == 0:
                comm_br[0, :, :] = compute_q(my_pos, OFF_RB).astype(jnp.bfloat16)
            if k == 1:
                comm_bl[0, :, :] = compute_q(my_pos, OFF_LB).astype(jnp.bfloat16)
            rdma_r.wait()
            rdma_l.wait()
            rs = (k + 1) % 2
            acc_r = comm_ar[rs, :, :].astype(jnp.float32) + p_r
            acc_l = comm_al[rs, :, :].astype(jnp.float32) + p_l
            if k < N_DEV - 2:
                comm_ar[rs, :, :] = acc_r.astype(jnp.bfloat16)
                comm_al[rs, :, :] = acc_l.astype(jnp.bfloat16)
            else:
                out_ref[pl.ds(c_r * ch + OFF_RA, q), :] = acc_r.astype(jnp.bfloat16)
                out_ref[pl.ds(c_l * ch + OFF_LA, q), :] = acc_l.astype(jnp.bfloat16)

        for j in range(N_DEV - 1):
            k = j + N_DEV - 1
            c_fr = (my_pos + 1 + N_DEV - j) % N_DEV
            c_fl = (my_pos + N_DEV - 1 + j) % N_DEV
            ag_r = ag_hop(c_fr, OFF_RA, send_ar, recv_ar, k, right)
            ag_l = ag_hop(c_fl, OFF_LA, send_al, recv_al, k, left)
            rdma_r = rs_hop(comm_br, send_br, recv_br, j, right)
            rdma_l = rs_hop(comm_bl, send_bl, recv_bl, j, left)
            ag_r.start()
            ag_l.start()
            rdma_r.start()
            rdma_l.start()
            c_r = (my_pos + (N_DEV - 1 - j)) % N_DEV
            c_l = (my_pos + j + 1) % N_DEV
            p_r = compute_q(c_r, OFF_RB)
            p_l = compute_q(c_l, OFF_LB)
            rdma_r.wait()
            rdma_l.wait()
            rs = (j + 1) % 2
            acc_r = comm_br[rs, :, :].astype(jnp.float32) + p_r
            acc_l = comm_bl[rs, :, :].astype(jnp.float32) + p_l
            if j < N_DEV - 2:
                comm_br[rs, :, :] = acc_r.astype(jnp.bfloat16)
                comm_bl[rs, :, :] = acc_l.astype(jnp.bfloat16)
            else:
                out_ref[pl.ds(c_r * ch + OFF_RB, q), :] = acc_r.astype(jnp.bfloat16)
                out_ref[pl.ds(c_l * ch + OFF_LB, q), :] = acc_l.astype(jnp.bfloat16)
            ag_r.wait()
            ag_l.wait()

        for t in range(N_DEV - 1):
            k = t + N_DEV - 1
            c_fr = (my_pos + 1 + N_DEV - t) % N_DEV
            c_fl = (my_pos + N_DEV - 1 + t) % N_DEV
            ag_r = ag_hop(c_fr, OFF_RB, send_br, recv_br, k, right)
            ag_l = ag_hop(c_fl, OFF_LB, send_bl, recv_bl, k, left)
            ag_r.start()
            ag_l.start()
            ag_r.wait()
            ag_l.wait()

    return pl.pallas_call(
        body,
        out_shape=jax.ShapeDtypeStruct((m, out_n), jnp.bfloat16),
        in_specs=[
            pl.BlockSpec(memory_space=pltpu.VMEM),
            pl.BlockSpec(memory_space=pltpu.VMEM),
            pl.BlockSpec(memory_space=pltpu.VMEM),
        ],
        out_specs=pl.BlockSpec(memory_space=pltpu.VMEM),
        scratch_shapes=[
            pltpu.VMEM((2, q, out_n), jnp.bfloat16),
            pltpu.VMEM((2, q, out_n), jnp.bfloat16),
            pltpu.VMEM((2, q, out_n), jnp.bfloat16),
            pltpu.VMEM((2, q, out_n), jnp.bfloat16),
            pltpu.SemaphoreType.DMA((2,)),
            pltpu.SemaphoreType.DMA((2,)),
            pltpu.SemaphoreType.DMA((2,)),
            pltpu.SemaphoreType.DMA((2,)),
            pltpu.SemaphoreType.DMA((2,)),
            pltpu.SemaphoreType.DMA((2,)),
            pltpu.SemaphoreType.DMA((2,)),
            pltpu.SemaphoreType.DMA((2,)),
        ],
        compiler_params=pltpu.CompilerParams(
            collective_id=0,
            vmem_limit_bytes=128 * 1024 * 1024,
        ),
    )(x, W1, W2)


# device time: 87566 ns/iter; 1.2565x vs baseline; 1.2565x over previous
import jax
import jax.numpy as jnp
from jax import lax
from jax.experimental import pallas as pl
from jax.experimental.pallas import tpu as pltpu

N_DEV = 4


def kernel(x, W1, W2):
    m, k_in = x.shape
    h_n = W1.shape[1]
    out_n = W2.shape[1]
    ch = m // N_DEV
    half = ch // 2

    NBLK = 8
    w1_blk = k_in // NBLK
    w2_blk = h_n // NBLK

    def body(x_ref, w1_hbm, w2_hbm, out_ref,
             w1b, w2b, stg1, stg2, comm_r, comm_l,
             stg1_sem, stg2_sem,
             send_r, recv_r, send_l, recv_l):
        my_pos = lax.axis_index("i")
        left = (my_pos + N_DEV - 1) % N_DEV
        right = (my_pos + 1) % N_DEV

        def w1_dma(b):
            return pltpu.make_async_copy(
                w1_hbm.at[pl.ds(b * w1_blk, w1_blk), :],
                stg1.at[b % 2], stg1_sem.at[b % 2])

        def w2_dma(b):
            return pltpu.make_async_copy(
                w2_hbm.at[pl.ds(b * w2_blk, w2_blk), :],
                stg2.at[b % 2], stg2_sem.at[b % 2])

        w1_dma(0).start()
        w2_dma(0).start()
        for b in range(NBLK):
            if b + 1 < NBLK:
                w1_dma(b + 1).start()
            w1_dma(b).wait()
            w1b[pl.ds(b * w1_blk, w1_blk), :] = (
                stg1[b % 2].astype(jnp.bfloat16))
        for b in range(NBLK):
            if b + 1 < NBLK:
                w2_dma(b + 1).start()
            w2_dma(b).wait()
            w2b[pl.ds(b * w2_blk, w2_blk), :] = (
                stg2[b % 2].astype(jnp.bfloat16))

        def compute_half(c, off):
            xs = x_ref[pl.ds(c * ch + off, half), :].astype(jnp.bfloat16)
            h = jnp.dot(xs, w1b[...], preferred_element_type=jnp.float32)
            h = jnp.maximum(h, 0.0).astype(jnp.bfloat16)
            return jnp.dot(h, w2b[...], preferred_element_type=jnp.float32)

        comm_r[0, :, :] = compute_half(my_pos, 0).astype(jnp.bfloat16)
        comm_l[0, :, :] = compute_half(my_pos, half).astype(jnp.bfloat16)

        barrier_sem = pltpu.get_barrier_semaphore()
        for nbr in [left, right]:
            pl.semaphore_signal(
                barrier_sem, inc=1,
                device_id=(nbr,), device_id_type=pl.DeviceIdType.MESH,
            )
        pl.semaphore_wait(barrier_sem, 2)

        for k in range(N_DEV - 1):
            ss, rs = k % 2, (k + 1) % 2
            rdma_r = pltpu.make_async_remote_copy(
                src_ref=comm_r.at[ss], dst_ref=comm_r.at[rs],
                send_sem=send_r.at[ss], recv_sem=recv_r.at[rs],
                device_id=(right,), device_id_type=pl.DeviceIdType.MESH,
            )
            rdma_l = pltpu.make_async_remote_copy(
                src_ref=comm_l.at[ss], dst_ref=comm_l.at[rs],
                send_sem=send_l.at[ss], recv_sem=recv_l.at[rs],
                device_id=(left,), device_id_type=pl.DeviceIdType.MESH,
            )
            rdma_r.start()
            rdma_l.start()
            c_r = (my_pos + (N_DEV - 1 - k)) % N_DEV
            c_l = (my_pos + k + 1) % N_DEV
            p_r = compute_half(c_r, 0)
            p_l = compute_half(c_l, half)
            rdma_r.wait()
            rdma_l.wait()
            acc_r = comm_r[rs, :, :].astype(jnp.float32) + p_r
            acc_l = comm_l[rs, :, :].astype(jnp.float32) + p_l
            if k < N_DEV - 2:
                comm_r[rs, :, :] = acc_r.astype(jnp.bfloat16)
                comm_l[rs, :, :] = acc_l.astype(jnp.bfloat16)
            else:
                out_ref[pl.ds(c_r * ch, half), :] = acc_r.astype(jnp.bfloat16)
                out_ref[pl.ds(c_l * ch + half, half), :] = (
                    acc_l.astype(jnp.bfloat16))

        for t in range(N_DEV - 1):
            k = t + N_DEV - 1
            ss, rs = k % 2, (k + 1) % 2
            c_fr = (my_pos + 1 + N_DEV - t) % N_DEV
            c_fl = (my_pos + N_DEV - 1 + t) % N_DEV
            reg_r = out_ref.at[pl.ds(c_fr * ch, half)]
            reg_l = out_ref.at[pl.ds(c_fl * ch + half, half)]
            rdma_r = pltpu.make_async_remote_copy(
                src_ref=reg_r, dst_ref=reg_r,
                send_sem=send_r.at[ss], recv_sem=recv_r.at[rs],
                device_id=(right,), device_id_type=pl.DeviceIdType.MESH,
            )
            rdma_l = pltpu.make_async_remote_copy(
                src_ref=reg_l, dst_ref=reg_l,
                send_sem=send_l.at[ss], recv_sem=recv_l.at[rs],
                device_id=(left,), device_id_type=pl.DeviceIdType.MESH,
            )
            rdma_r.start()
            rdma_l.start()
            rdma_r.wait()
            rdma_l.wait()

    return pl.pallas_call(
        body,
        out_shape=jax.ShapeDtypeStruct((m, out_n), jnp.bfloat16),
        in_specs=[
            pl.BlockSpec(memory_space=pltpu.VMEM),
            pl.BlockSpec(memory_space=pltpu.MemorySpace.HBM),
            pl.BlockSpec(memory_space=pltpu.MemorySpace.HBM),
        ],
        out_specs=pl.BlockSpec(memory_space=pltpu.VMEM),
        scratch_shapes=[
            pltpu.VMEM((k_in, h_n), jnp.bfloat16),
            pltpu.VMEM((h_n, out_n), jnp.bfloat16),
            pltpu.VMEM((2, w1_blk, h_n), jnp.float32),
            pltpu.VMEM((2, w2_blk, out_n), jnp.float32),
            pltpu.VMEM((2, half, out_n), jnp.bfloat16),
            pltpu.VMEM((2, half, out_n), jnp.bfloat16),
            pltpu.SemaphoreType.DMA((2,)),
            pltpu.SemaphoreType.DMA((2,)),
            pltpu.SemaphoreType.DMA((2,)),
            pltpu.SemaphoreType.DMA((2,)),
            pltpu.SemaphoreType.DMA((2,)),
            pltpu.SemaphoreType.DMA((2,)),
        ],
        compiler_params=pltpu.CompilerParams(
            collective_id=0,
            vmem_limit_bytes=128 * 1024 * 1024,
        ),
    )(x, W1, W2)


# device time: 86548 ns/iter; 1.2713x vs baseline; 1.0118x over previous
import jax
import jax.numpy as jnp
from jax import lax
from jax.experimental import pallas as pl
from jax.experimental.pallas import tpu as pltpu

N_DEV = 4


def kernel(x, W1, W2):
    m, k_in = x.shape
    h_n = W1.shape[1]
    out_n = W2.shape[1]
    ch = m // N_DEV
    half = ch // 2

    NBLK = 8
    w1_blk = k_in // NBLK
    w2_blk = h_n // NBLK

    def body(x_ref, w1_hbm, w2_hbm, out_ref,
             w1b, w2b, stg1, stg2, comm_r, comm_l,
             stg1_sem, stg2_sem,
             send_r, recv_r, send_l, recv_l):
        my_pos = lax.axis_index("i")
        left = (my_pos + N_DEV - 1) % N_DEV
        right = (my_pos + 1) % N_DEV

        def w1_dma(b):
            return pltpu.make_async_copy(
                w1_hbm.at[pl.ds(b * w1_blk, w1_blk), :],
                stg1.at[b % 2], stg1_sem.at[b % 2])

        def w2_dma(b):
            return pltpu.make_async_copy(
                w2_hbm.at[pl.ds(b * w2_blk, w2_blk), :],
                stg2.at[b % 2], stg2_sem.at[b % 2])

        xs_seed = x_ref[pl.ds(my_pos * ch, ch), :].astype(jnp.bfloat16)
        w1_dma(0).start()
        w2_dma(0).start()
        h_acc = jnp.zeros((ch, h_n), jnp.float32)
        for b in range(NBLK):
            if b + 1 < NBLK:
                w1_dma(b + 1).start()
            w1_dma(b).wait()
            wblk = stg1[b % 2].astype(jnp.bfloat16)
            w1b[pl.ds(b * w1_blk, w1_blk), :] = wblk
            h_acc = h_acc + jnp.dot(
                xs_seed[:, b * w1_blk:(b + 1) * w1_blk], wblk,
                preferred_element_type=jnp.float32)
        h_seed = jnp.maximum(h_acc, 0.0).astype(jnp.bfloat16)
        p_acc = jnp.zeros((ch, out_n), jnp.float32)
        for b in range(NBLK):
            if b + 1 < NBLK:
                w2_dma(b + 1).start()
            w2_dma(b).wait()
            wblk = stg2[b % 2].astype(jnp.bfloat16)
            w2b[pl.ds(b * w2_blk, w2_blk), :] = wblk
            p_acc = p_acc + jnp.dot(
                h_seed[:, b * w2_blk:(b + 1) * w2_blk], wblk,
                preferred_element_type=jnp.float32)

        def compute_half(c, off):
            xs = x_ref[pl.ds(c * ch + off, half), :].astype(jnp.bfloat16)
            h = jnp.dot(xs, w1b[...], preferred_element_type=jnp.float32)
            h = jnp.maximum(h, 0.0).astype(jnp.bfloat16)
            return jnp.dot(h, w2b[...], preferred_element_type=jnp.float32)

        comm_r[0, :, :] = p_acc[:half, :].astype(jnp.bfloat16)
        comm_l[0, :, :] = p_acc[half:, :].astype(jnp.bfloat16)

        barrier_sem = pltpu.get_barrier_semaphore()
        for nbr in [left, right]:
            pl.semaphore_signal(
                barrier_sem, inc=1,
                device_id=(nbr,), device_id_type=pl.DeviceIdType.MESH,
            )
        pl.semaphore_wait(barrier_sem, 2)

        for k in range(N_DEV - 1):
            ss, rs = k % 2, (k + 1) % 2
            rdma_r = pltpu.make_async_remote_copy(
                src_ref=comm_r.at[ss], dst_ref=comm_r.at[rs],
                send_sem=send_r.at[ss], recv_sem=recv_r.at[rs],
                device_id=(right,), device_id_type=pl.DeviceIdType.MESH,
            )
            rdma_l = pltpu.make_async_remote_copy(
                src_ref=comm_l.at[ss], dst_ref=comm_l.at[rs],
                send_sem=send_l.at[ss], recv_sem=recv_l.at[rs],
                device_id=(left,), device_id_type=pl.DeviceIdType.MESH,
            )
            rdma_r.start()
            rdma_l.start()
            c_r = (my_pos + (N_DEV - 1 - k)) % N_DEV
            c_l = (my_pos + k + 1) % N_DEV
            p_r = compute_half(c_r, 0)
            p_l = compute_half(c_l, half)
            rdma_r.wait()
            rdma_l.wait()
            acc_r = comm_r[rs, :, :].astype(jnp.float32) + p_r
            acc_l = comm_l[rs, :, :].astype(jnp.float32) + p_l
            if k < N_DEV - 2:
                comm_r[rs, :, :] = acc_r.astype(jnp.bfloat16)
                comm_l[rs, :, :] = acc_l.astype(jnp.bfloat16)
            else:
                out_ref[pl.ds(c_r * ch, half), :] = acc_r.astype(jnp.bfloat16)
                out_ref[pl.ds(c_l * ch + half, half), :] = (
                    acc_l.astype(jnp.bfloat16))

        for t in range(N_DEV - 1):
            k = t + N_DEV - 1
            ss, rs = k % 2, (k + 1) % 2
            c_fr = (my_pos + 1 + N_DEV - t) % N_DEV
            c_fl = (my_pos + N_DEV - 1 + t) % N_DEV
            reg_r = out_ref.at[pl.ds(c_fr * ch, half)]
            reg_l = out_ref.at[pl.ds(c_fl * ch + half, half)]
            rdma_r = pltpu.make_async_remote_copy(
                src_ref=reg_r, dst_ref=reg_r,
                send_sem=send_r.at[ss], recv_sem=recv_r.at[rs],
                device_id=(right,), device_id_type=pl.DeviceIdType.MESH,
            )
            rdma_l = pltpu.make_async_remote_copy(
                src_ref=reg_l, dst_ref=reg_l,
                send_sem=send_l.at[ss], recv_sem=recv_l.at[rs],
                device_id=(left,), device_id_type=pl.DeviceIdType.MESH,
            )
            rdma_r.start()
            rdma_l.start()
            rdma_r.wait()
            rdma_l.wait()

    return pl.pallas_call(
        body,
        out_shape=jax.ShapeDtypeStruct((m, out_n), jnp.bfloat16),
        in_specs=[
            pl.BlockSpec(memory_space=pltpu.VMEM),
            pl.BlockSpec(memory_space=pltpu.MemorySpace.HBM),
            pl.BlockSpec(memory_space=pltpu.MemorySpace.HBM),
        ],
        out_specs=pl.BlockSpec(memory_space=pltpu.VMEM),
        scratch_shapes=[
            pltpu.VMEM((k_in, h_n), jnp.bfloat16),
            pltpu.VMEM((h_n, out_n), jnp.bfloat16),
            pltpu.VMEM((2, w1_blk, h_n), jnp.float32),
            pltpu.VMEM((2, w2_blk, out_n), jnp.float32),
            pltpu.VMEM((2, half, out_n), jnp.bfloat16),
            pltpu.VMEM((2, half, out_n), jnp.bfloat16),
            pltpu.SemaphoreType.DMA((2,)),
            pltpu.SemaphoreType.DMA((2,)),
            pltpu.SemaphoreType.DMA((2,)),
            pltpu.SemaphoreType.DMA((2,)),
            pltpu.SemaphoreType.DMA((2,)),
            pltpu.SemaphoreType.DMA((2,)),
        ],
        compiler_params=pltpu.CompilerParams(
            collective_id=0,
            vmem_limit_bytes=128 * 1024 * 1024,
        ),
    )(x, W1, W2)
